# baseline (speedup 1.0000x reference)
import functools
import math

import jax
import jax.numpy as jnp
from jax import lax
from jax.experimental import pallas as pl
from jax.experimental.pallas import tpu as pltpu
from jax.experimental.pallas import tpu_sc as plsc


_BS = 2048
_SC_ROWS = 2048

_INV = 0.6366197723675814
_C1 = 1.5703125
_C2 = math.pi / 2 - 1.5703125
_S1, _S2, _S3 = -0.16666667, 0.0083333310, -1.9840874e-4
_CA, _CB, _CC = 0.041666638, -0.0013888380, 2.4760127e-5


def _make_sc_gather(S, V, D):
    info = plsc.get_sparse_core_info()
    NC, NS = info.num_cores, info.num_subcores
    NW = NC * NS
    b_per_w = S // NW
    mesh = plsc.VectorSubcoreMesh(core_axis_name="c", subcore_axis_name="s")

    @functools.partial(
        pl.kernel,
        mesh=mesh,
        out_type=jax.ShapeDtypeStruct((S, D), jnp.float32),
        scratch_types=[
            pltpu.VMEM((b_per_w,), jnp.int32),
            pltpu.VMEM((b_per_w, D), jnp.float32),
            pltpu.SemaphoreType.DMA,
        ],
    )
    def k(t_hbm, pe_hbm, out_hbm, idx_v, rows_v, sem):
        wid = lax.axis_index("s") * NC + lax.axis_index("c")
        base = wid * b_per_w
        pltpu.sync_copy(t_hbm.at[pl.ds(base, b_per_w)], idx_v)
        pltpu.async_copy(pe_hbm.at[idx_v], rows_v, sem).wait()
        pltpu.sync_copy(rows_v, out_hbm.at[pl.ds(base, b_per_w)])

    return k


def _tc_body(t_ref, d_ref, o_ref):
    bs, D = o_ref.shape
    t = t_ref[0, 0, :].astype(jnp.float32)[:, None]
    d = d_ref[0, :][None, :]
    x = t * d
    q = jnp.floor(x * _INV + 0.5)
    qi = q.astype(jnp.int32)
    r = (x - q * _C1) - q * _C2
    r2 = r * r
    s = r * (1.0 + r2 * (_S1 + r2 * (_S2 + r2 * _S3)))
    c = 1.0 - 0.5 * r2 + r2 * r2 * (_CA + r2 * (_CB + r2 * _CC))
    col = lax.broadcasted_iota(jnp.int32, (bs, D), 1)
    m = (qi + (col & 1)) & 3
    val = jnp.where((m & 1) == 1, c, s)
    o_ref[...] = jnp.where((m & 2) == 2, -val, val)


def kernel(t, pe):
    (B,) = t.shape
    V, D = pe.shape
    t = t.astype(jnp.int32)

    sc_part = _make_sc_gather(_SC_ROWS, V, D)(t[:_SC_ROWS], pe)

    nb = B // _BS
    sb = _SC_ROWS // _BS
    t3 = t.reshape(nb, 1, _BS)
    div = jnp.exp(jnp.arange(0, D, 2, dtype=jnp.float32) * (-math.log(10000.0) / D))
    dcol = jnp.repeat(div, 2).reshape(1, D)
    tc_full = pl.pallas_call(
        _tc_body,
        grid=(nb - sb,),
        in_specs=[
            pl.BlockSpec((1, 1, _BS), lambda i: (i + sb, 0, 0)),
            pl.BlockSpec((1, D), lambda i: (0, 0)),
        ],
        out_specs=pl.BlockSpec((_BS, D), lambda i: (i + sb, 0)),
        out_shape=jax.ShapeDtypeStruct((B, D), jnp.float32),
    )(t3, dcol)

    return lax.dynamic_update_slice(tc_full, sc_part, (0, 0))


# speedup vs baseline: 1.1833x; 1.1833x over previous
import functools

import jax
import jax.numpy as jnp
from jax import lax
from jax.experimental import pallas as pl
from jax.experimental.pallas import tpu as pltpu
from jax.experimental.pallas import tpu_sc as plsc


def _make_gather(B, V, D):
    info = plsc.get_sparse_core_info()
    NC, NS = info.num_cores, info.num_subcores
    NW = NC * NS
    b_per_w = B // NW
    mesh = plsc.VectorSubcoreMesh(core_axis_name="c", subcore_axis_name="s")

    @functools.partial(
        pl.kernel,
        mesh=mesh,
        out_type=jax.ShapeDtypeStruct((B, D), jnp.float32),
        scratch_types=[
            pltpu.VMEM((b_per_w,), jnp.int32),
            pltpu.VMEM((b_per_w, D), jnp.float32),
            pltpu.SemaphoreType.DMA,
        ],
    )
    def k(t_hbm, pe_hbm, out_hbm, idx_v, rows_v, sem):
        wid = lax.axis_index("s") * NC + lax.axis_index("c")
        base = wid * b_per_w
        pltpu.sync_copy(t_hbm.at[pl.ds(base, b_per_w)], idx_v)
        pltpu.async_copy(pe_hbm.at[idx_v], rows_v, sem).wait()
        pltpu.sync_copy(rows_v, out_hbm.at[pl.ds(base, b_per_w)])

    return k


def kernel(t, pe):
    (B,) = t.shape
    V, D = pe.shape
    fn = _make_gather(B, V, D)
    return fn(t.astype(jnp.int32), pe.astype(jnp.float32))
